# baseline (speedup 1.0000x reference)
import numpy as np
import jax
import jax.numpy as jnp
from jax.experimental import pallas as pl
from jax.experimental.pallas import tpu as pltpu

L = 2048
H = 128
F = 4
K = L // 2


def _build_dft_consts():
    t = np.arange(L, dtype=np.float64)
    k = np.arange(K, dtype=np.float64)
    ang = (2.0 * np.pi / L) * np.outer(k, t)
    cf = (np.cos(ang) / L).astype(np.float32)
    sf = (-np.sin(ang) / L).astype(np.float32)
    alpha = np.where(k == 0, 1.0, 2.0)[:, None]
    ci = (alpha * np.cos(ang)).T.astype(np.float32)
    si = (-(alpha * np.sin(ang))).T.astype(np.float32)
    return (cf.astype(jnp.bfloat16), sf.astype(jnp.bfloat16),
            ci.astype(jnp.bfloat16), si.astype(jnp.bfloat16))


_CF, _SF, _CI, _SI = _build_dft_consts()


def _sigmoid(x):
    return 1.0 / (1.0 + jnp.exp(-x))


def _stg_body(x1_ref, x2_ref, x3_ref, x4_ref, cf_ref, sf_ref, ci_ref,
              si_ref, cwr_ref, cwi_ref, cwn_ref, w1_ref, b1_ref, w2t_ref,
              b2_ref, g1_ref, be1_ref, g2_ref, be2_ref, oa_ref, ob_ref):
    x1 = x1_ref[0]
    x2 = x2_ref[0]
    x3 = x3_ref[0]
    x4 = x4_ref[0]

    m3 = jnp.mean(x3, axis=0, keepdims=True)
    m4 = jnp.mean(x4, axis=0, keepdims=True)
    X = jnp.concatenate([m3, m4, m3, m4], axis=1)
    mu = jnp.mean(X, axis=1, keepdims=True)
    var = jnp.mean((X - mu) ** 2, axis=1, keepdims=True)
    Xn = (X - mu) * jax.lax.rsqrt(var + 1e-5) * g1_ref[...] + be1_ref[...]
    h = jnp.dot(Xn, w1_ref[...], preferred_element_type=jnp.float32)
    h = jnp.maximum(h + b1_ref[...], 0.0)
    logits = jnp.sum(h * w2t_ref[...], axis=1, keepdims=True) + b2_ref[...]
    lm = jnp.max(logits, axis=0, keepdims=True)
    e = jnp.exp(logits - lm)
    fw = e / jnp.sum(e, axis=0, keepdims=True)
    fw3 = fw.reshape(F, 1, 1)

    wr = _sigmoid(jnp.sum(fw3 * cwr_ref[...], axis=0))
    wi = _sigmoid(jnp.sum(fw3 * cwi_ref[...], axis=0))
    cwn = jnp.sum(fw3 * cwn_ref[...], axis=0)
    wrn = _sigmoid(cwn[0:1])

    xcat = jnp.concatenate([x1, x2], axis=1).astype(jnp.bfloat16)
    RE = jnp.dot(cf_ref[...], xcat, preferred_element_type=jnp.float32)
    IM = jnp.dot(sf_ref[...], xcat, preferred_element_type=jnp.float32)
    re1, re2 = RE[:, :H], RE[:, H:]
    im1, im2 = IM[:, :H], IM[:, H:]

    fr1 = re1 * wr - im1 * wi
    fi1 = re1 * wi + im1 * wr
    wrb = 1.0 - wr
    fr2 = re2 * wrb + im2 * wi
    fi2 = im2 * wrb - re2 * wi
    Fr = jnp.concatenate([fr1, fr2], axis=1).astype(jnp.bfloat16)
    Fi = jnp.concatenate([fi1, fi2], axis=1).astype(jnp.bfloat16)

    Y = (jnp.dot(ci_ref[...], Fr, preferred_element_type=jnp.float32)
         + jnp.dot(si_ref[...], Fi, preferred_element_type=jnp.float32))

    tidx = jax.lax.broadcasted_iota(jnp.int32, (L, 1), 0)
    alt = jnp.where((tidx % 2) == 0, 1.0, -1.0).astype(jnp.float32)
    ren1 = jnp.sum(x1 * alt, axis=0, keepdims=True) * (1.0 / L)
    ren2 = jnp.sum(x2 * alt, axis=0, keepdims=True) * (1.0 / L)
    ya = Y[:, :H] + alt * (ren1 * wrn)
    yb = Y[:, H:] + alt * (ren2 * (1.0 - wrn))

    g2 = g2_ref[...]
    be2 = be2_ref[...]
    sa = ya + x1
    mua = jnp.mean(sa, axis=1, keepdims=True)
    va = jnp.mean((sa - mua) ** 2, axis=1, keepdims=True)
    oa_ref[...] = (((sa - mua) * jax.lax.rsqrt(va + 1e-5)) * g2 + be2)[None]
    sb = yb + x2
    mub = jnp.mean(sb, axis=1, keepdims=True)
    vb = jnp.mean((sb - mub) ** 2, axis=1, keepdims=True)
    ob_ref[...] = (((sb - mub) * jax.lax.rsqrt(vb + 1e-5)) * g2 + be2)[None]


def kernel(input_tensor1, input_tensor2, input_tensor3, input_tensor4,
           complex_weight, W1, b1, W2, b2, ln1_g, ln1_b, ln2_g, ln2_b):
    B = input_tensor1.shape[0]

    cw = jnp.transpose(complex_weight[0], (2, 3, 0, 1))
    cwr = cw[:, 0, :K, :]
    cwi = cw[:, 1, :K, :]
    cwn = cw[:, :, K, :]

    batch_in = lambda b: (b, 0, 0)
    const2 = lambda b: (0, 0)
    const3 = lambda b: (0, 0, 0)

    grid_spec = pl.GridSpec(
        grid=(B,),
        in_specs=[
            pl.BlockSpec((1, L, H), batch_in),
            pl.BlockSpec((1, L, H), batch_in),
            pl.BlockSpec((1, L, H // 2), batch_in),
            pl.BlockSpec((1, L, H // 2), batch_in),
            pl.BlockSpec((K, L), const2),
            pl.BlockSpec((K, L), const2),
            pl.BlockSpec((L, K), const2),
            pl.BlockSpec((L, K), const2),
            pl.BlockSpec((F, K, H), const3),
            pl.BlockSpec((F, K, H), const3),
            pl.BlockSpec((F, 2, H), const3),
            pl.BlockSpec((2 * H, H), const2),
            pl.BlockSpec((1, H), const2),
            pl.BlockSpec((F, H), const2),
            pl.BlockSpec((F, 1), const2),
            pl.BlockSpec((1, 2 * H), const2),
            pl.BlockSpec((1, 2 * H), const2),
            pl.BlockSpec((1, H), const2),
            pl.BlockSpec((1, H), const2),
        ],
        out_specs=[
            pl.BlockSpec((1, L, H), batch_in),
            pl.BlockSpec((1, L, H), batch_in),
        ],
    )

    out_a, out_b = pl.pallas_call(
        _stg_body,
        grid_spec=grid_spec,
        out_shape=[
            jax.ShapeDtypeStruct((B, L, H), jnp.float32),
            jax.ShapeDtypeStruct((B, L, H), jnp.float32),
        ],
        compiler_params=pltpu.CompilerParams(
            dimension_semantics=("parallel",),
        ),
    )(
        input_tensor1, input_tensor2, input_tensor3, input_tensor4,
        jnp.asarray(_CF), jnp.asarray(_SF), jnp.asarray(_CI),
        jnp.asarray(_SI), cwr, cwi, cwn,
        W1, b1.reshape(1, H), W2.T, b2.reshape(F, 1),
        ln1_g.reshape(1, 2 * H), ln1_b.reshape(1, 2 * H),
        ln2_g.reshape(1, H), ln2_b.reshape(1, H),
    )
    return (out_a, out_b)


# speedup vs baseline: 1.3676x; 1.3676x over previous
import numpy as np
import jax
import jax.numpy as jnp
from jax.experimental import pallas as pl
from jax.experimental.pallas import tpu as pltpu

L = 2048
H = 128
F = 4
K = L // 2
M = K // 2


def _build_dft_consts():
    t = np.arange(K, dtype=np.float64)
    m = np.arange(M, dtype=np.float64)
    ang_e = (2.0 * np.pi / K) * np.outer(m, t)
    ang_o = (2.0 * np.pi / L) * np.outer(2 * m + 1, t)
    cse = np.concatenate([np.cos(ang_e), -np.sin(ang_e)], axis=0) / L
    cso = np.concatenate([np.cos(ang_o), -np.sin(ang_o)], axis=0) / L
    alpha_e = np.where(m == 0, 1.0, 2.0)[None, :]
    cie = np.concatenate([alpha_e * np.cos(ang_e.T),
                          -alpha_e * np.sin(ang_e.T)], axis=1)
    cio = np.concatenate([2.0 * np.cos(ang_o.T),
                          -2.0 * np.sin(ang_o.T)], axis=1)
    return (cse.astype(np.float32).astype(jnp.bfloat16),
            cso.astype(np.float32).astype(jnp.bfloat16),
            cie.astype(np.float32).astype(jnp.bfloat16),
            cio.astype(np.float32).astype(jnp.bfloat16))


_CSE, _CSO, _CIE, _CIO = _build_dft_consts()


def _sigmoid(x):
    return 1.0 / (1.0 + jnp.exp(-x))


def _stg_body(x1_ref, x2_ref, x3_ref, x4_ref, cse_ref, cso_ref, cie_ref,
              cio_ref, cwr_ref, cwi_ref, cwn_ref, w1_ref, b1_ref, w2t_ref,
              b2_ref, g1_ref, be1_ref, g2_ref, be2_ref, oa_ref, ob_ref):
    x1 = x1_ref[0]
    x2 = x2_ref[0]
    x3 = x3_ref[0]
    x4 = x4_ref[0]

    m3 = jnp.mean(x3, axis=0, keepdims=True)
    m4 = jnp.mean(x4, axis=0, keepdims=True)
    X = jnp.concatenate([m3, m4, m3, m4], axis=1)
    mu = jnp.mean(X, axis=1, keepdims=True)
    var = jnp.mean((X - mu) ** 2, axis=1, keepdims=True)
    Xn = (X - mu) * jax.lax.rsqrt(var + 1e-5) * g1_ref[...] + be1_ref[...]
    h = jnp.dot(Xn, w1_ref[...], preferred_element_type=jnp.float32)
    h = jnp.maximum(h + b1_ref[...], 0.0)
    logits = jnp.sum(h * w2t_ref[...], axis=1, keepdims=True) + b2_ref[...]
    lm = jnp.max(logits, axis=0, keepdims=True)
    e = jnp.exp(logits - lm)
    fw = e / jnp.sum(e, axis=0, keepdims=True)
    fw3 = fw.reshape(F, 1, 1)

    wr = _sigmoid(jnp.sum(fw3 * cwr_ref[...], axis=0))
    wi = _sigmoid(jnp.sum(fw3 * cwi_ref[...], axis=0))
    cwn = jnp.sum(fw3 * cwn_ref[...], axis=0)
    wrn = _sigmoid(cwn[0:1])

    xcat = jnp.concatenate([x1, x2], axis=1)
    u = (xcat[:K] + xcat[K:]).astype(jnp.bfloat16)
    v = (xcat[:K] - xcat[K:]).astype(jnp.bfloat16)
    ME = jnp.dot(cse_ref[...], u, preferred_element_type=jnp.float32)
    MO = jnp.dot(cso_ref[...], v, preferred_element_type=jnp.float32)

    def gate(Mat, wr_c, wi_c):
        re1, re2 = Mat[:M, :H], Mat[:M, H:]
        im1, im2 = Mat[M:, :H], Mat[M:, H:]
        wrb = 1.0 - wr_c
        fr = jnp.concatenate([re1 * wr_c - im1 * wi_c,
                              re2 * wrb + im2 * wi_c], axis=1)
        fi = jnp.concatenate([re1 * wi_c + im1 * wr_c,
                              im2 * wrb - re2 * wi_c], axis=1)
        return jnp.concatenate([fr, fi], axis=0).astype(jnp.bfloat16)

    GE = gate(ME, wr[:M], wi[:M])
    GO = gate(MO, wr[M:], wi[M:])

    P = jnp.dot(cie_ref[...], GE, preferred_element_type=jnp.float32)
    Q = jnp.dot(cio_ref[...], GO, preferred_element_type=jnp.float32)
    Y = jnp.concatenate([P + Q, P - Q], axis=0)

    tidx = jax.lax.broadcasted_iota(jnp.int32, (L, 1), 0)
    alt = jnp.where((tidx % 2) == 0, 1.0, -1.0).astype(jnp.float32)
    ren1 = jnp.sum(x1 * alt, axis=0, keepdims=True) * (1.0 / L)
    ren2 = jnp.sum(x2 * alt, axis=0, keepdims=True) * (1.0 / L)
    ya = Y[:, :H] + alt * (ren1 * wrn)
    yb = Y[:, H:] + alt * (ren2 * (1.0 - wrn))

    g2 = g2_ref[...]
    be2 = be2_ref[...]
    sa = ya + x1
    mua = jnp.mean(sa, axis=1, keepdims=True)
    va = jnp.mean((sa - mua) ** 2, axis=1, keepdims=True)
    oa_ref[...] = (((sa - mua) * jax.lax.rsqrt(va + 1e-5)) * g2 + be2)[None]
    sb = yb + x2
    mub = jnp.mean(sb, axis=1, keepdims=True)
    vb = jnp.mean((sb - mub) ** 2, axis=1, keepdims=True)
    ob_ref[...] = (((sb - mub) * jax.lax.rsqrt(vb + 1e-5)) * g2 + be2)[None]


def kernel(input_tensor1, input_tensor2, input_tensor3, input_tensor4,
           complex_weight, W1, b1, W2, b2, ln1_g, ln1_b, ln2_g, ln2_b):
    B = input_tensor1.shape[0]

    cw = jnp.transpose(complex_weight[0], (2, 3, 0, 1))
    cwr = jnp.concatenate([cw[:, 0, 0:K:2, :], cw[:, 0, 1:K:2, :]], axis=1)
    cwi = jnp.concatenate([cw[:, 1, 0:K:2, :], cw[:, 1, 1:K:2, :]], axis=1)
    cwn = cw[:, :, K, :]

    batch_in = lambda b: (b, 0, 0)
    const2 = lambda b: (0, 0)
    const3 = lambda b: (0, 0, 0)

    grid_spec = pl.GridSpec(
        grid=(B,),
        in_specs=[
            pl.BlockSpec((1, L, H), batch_in),
            pl.BlockSpec((1, L, H), batch_in),
            pl.BlockSpec((1, L, H // 2), batch_in),
            pl.BlockSpec((1, L, H // 2), batch_in),
            pl.BlockSpec((K, K), const2),
            pl.BlockSpec((K, K), const2),
            pl.BlockSpec((K, K), const2),
            pl.BlockSpec((K, K), const2),
            pl.BlockSpec((F, K, H), const3),
            pl.BlockSpec((F, K, H), const3),
            pl.BlockSpec((F, 2, H), const3),
            pl.BlockSpec((2 * H, H), const2),
            pl.BlockSpec((1, H), const2),
            pl.BlockSpec((F, H), const2),
            pl.BlockSpec((F, 1), const2),
            pl.BlockSpec((1, 2 * H), const2),
            pl.BlockSpec((1, 2 * H), const2),
            pl.BlockSpec((1, H), const2),
            pl.BlockSpec((1, H), const2),
        ],
        out_specs=[
            pl.BlockSpec((1, L, H), batch_in),
            pl.BlockSpec((1, L, H), batch_in),
        ],
    )

    out_a, out_b = pl.pallas_call(
        _stg_body,
        grid_spec=grid_spec,
        out_shape=[
            jax.ShapeDtypeStruct((B, L, H), jnp.float32),
            jax.ShapeDtypeStruct((B, L, H), jnp.float32),
        ],
        compiler_params=pltpu.CompilerParams(
            dimension_semantics=("arbitrary",),
        ),
    )(
        input_tensor1, input_tensor2, input_tensor3, input_tensor4,
        jnp.asarray(_CSE), jnp.asarray(_CSO), jnp.asarray(_CIE),
        jnp.asarray(_CIO), cwr, cwi, cwn,
        W1, b1.reshape(1, H), W2.T, b2.reshape(F, 1),
        ln1_g.reshape(1, 2 * H), ln1_b.reshape(1, 2 * H),
        ln2_g.reshape(1, H), ln2_b.reshape(1, H),
    )
    return (out_a, out_b)
